# baseline (speedup 1.0000x reference)
import jax
import jax.numpy as jnp
from jax.experimental import pallas as pl
from jax.experimental.pallas import tpu as pltpu

_S = 196
_P = 224 * 224
_KB = 3584
_NK = _P // _KB


def _sp_mean_kernel(mask_ref, xa_ref, wf_ref, out_ref, acc_ref):
    k = pl.program_id(1)

    @pl.when(k == 0)
    def _init():
        acc_ref[...] = jnp.zeros_like(acc_ref)

    m = mask_ref[0].astype(jnp.float32)
    xa = xa_ref[0]
    acc_ref[...] += jax.lax.dot_general(
        m, xa, (((1,), (1,)), ((), ())), preferred_element_type=jnp.float32)

    @pl.when(k == _NK - 1)
    def _finish():
        acc = acc_ref[...]
        counts = acc[:, 3:4]
        proj = jax.lax.dot_general(
            acc, wf_ref[...], (((1,), (0,)), ((), ())),
            preferred_element_type=jnp.float32)
        out_ref[0] = proj / counts


def kernel(X, masks, W, b):
    B = X.shape[0]
    Xf = X.reshape(B, 3, _P)
    ones = jnp.ones((B, 1, _P), jnp.float32)
    Xa = jnp.concatenate([Xf, ones], axis=1)
    Wf = jnp.concatenate([W.T, b[None, :]], axis=0)
    masks_r = masks.reshape(B, _S, _P)

    out = pl.pallas_call(
        _sp_mean_kernel,
        grid=(B, _NK),
        in_specs=[
            pl.BlockSpec((1, _S, _KB), lambda bi, ki: (bi, 0, ki)),
            pl.BlockSpec((1, 4, _KB), lambda bi, ki: (bi, 0, ki)),
            pl.BlockSpec((4, 56), lambda bi, ki: (0, 0)),
        ],
        out_specs=pl.BlockSpec((1, _S, 56), lambda bi, ki: (bi, 0, 0)),
        out_shape=jax.ShapeDtypeStruct((B, _S, 56), jnp.float32),
        scratch_shapes=[pltpu.VMEM((_S, 4), jnp.float32)],
    )(masks_r, Xa, Wf)
    return out


# speedup vs baseline: 9.2657x; 9.2657x over previous
import jax
import jax.numpy as jnp
from jax.experimental import pallas as pl
from jax.experimental.pallas import tpu as pltpu

_S = 196
_H = 224
_W = 224
_HB = 32
_NH = _H // _HB
_KB = _HB * _W


def _sp_mean_kernel(mask_ref, xa_ref, wf_ref, out_ref, acc_ref):
    k = pl.program_id(1)

    @pl.when(k == 0)
    def _init():
        acc_ref[...] = jnp.zeros_like(acc_ref)

    m = mask_ref[0].astype(jnp.int8).reshape(_S, _KB).astype(jnp.float32)
    xa = xa_ref[0].reshape(4, _KB)
    acc_ref[...] += jax.lax.dot_general(
        m, xa, (((1,), (1,)), ((), ())), preferred_element_type=jnp.float32)

    @pl.when(k == _NH - 1)
    def _finish():
        acc = acc_ref[...]
        counts = acc[:, 3:4]
        proj = jax.lax.dot_general(
            acc, wf_ref[...], (((1,), (0,)), ((), ())),
            preferred_element_type=jnp.float32)
        out_ref[0] = proj / counts


def kernel(X, masks, W, b):
    B = X.shape[0]
    ones = jnp.ones((B, 1, _H, _W), jnp.float32)
    Xa = jnp.concatenate([X, ones], axis=1)
    Wf = jnp.concatenate([W.T, b[None, :]], axis=0)

    out = pl.pallas_call(
        _sp_mean_kernel,
        grid=(B, _NH),
        in_specs=[
            pl.BlockSpec((1, _S, _HB, _W), lambda bi, ki: (bi, 0, ki, 0)),
            pl.BlockSpec((1, 4, _HB, _W), lambda bi, ki: (bi, 0, ki, 0)),
            pl.BlockSpec((4, 56), lambda bi, ki: (0, 0)),
        ],
        out_specs=pl.BlockSpec((1, _S, 56), lambda bi, ki: (bi, 0, 0)),
        out_shape=jax.ShapeDtypeStruct((B, _S, 56), jnp.float32),
        scratch_shapes=[pltpu.VMEM((_S, 4), jnp.float32)],
    )(masks, Xa, Wf)
    return out


# speedup vs baseline: 9.5132x; 1.0267x over previous
import jax
import jax.numpy as jnp
from jax.experimental import pallas as pl
from jax.experimental.pallas import tpu as pltpu

_S = 196
_H = 224
_W = 224
_HB = 32
_NH = _H // _HB
_KB = _HB * _W


def _sp_mean_kernel(mask_ref, xa_ref, wf_ref, out_ref, acc_ref):
    k = pl.program_id(1)

    @pl.when(k == 0)
    def _init():
        acc_ref[...] = jnp.zeros_like(acc_ref)

    m = mask_ref[0].astype(jnp.int8).reshape(_S, _KB).astype(jnp.float32)
    x3 = xa_ref[0].reshape(3, _KB)
    xa = jnp.concatenate([x3, jnp.ones((1, _KB), jnp.float32)], 0)
    acc_ref[...] += jax.lax.dot_general(
        m, xa, (((1,), (1,)), ((), ())), preferred_element_type=jnp.float32)

    @pl.when(k == _NH - 1)
    def _finish():
        acc = acc_ref[...]
        counts = acc[:, 3:4]
        proj = jax.lax.dot_general(
            acc, wf_ref[...], (((1,), (0,)), ((), ())),
            preferred_element_type=jnp.float32)
        out_ref[0] = proj / counts


def kernel(X, masks, W, b):
    B = X.shape[0]
    Wf = jnp.concatenate([W.T, b[None, :]], axis=0)

    out = pl.pallas_call(
        _sp_mean_kernel,
        grid=(B, _NH),
        in_specs=[
            pl.BlockSpec((1, _S, _HB, _W), lambda bi, ki: (bi, 0, ki, 0)),
            pl.BlockSpec((1, 3, _HB, _W), lambda bi, ki: (bi, 0, ki, 0)),
            pl.BlockSpec((4, 56), lambda bi, ki: (0, 0)),
        ],
        out_specs=pl.BlockSpec((1, _S, 56), lambda bi, ki: (bi, 0, 0)),
        out_shape=jax.ShapeDtypeStruct((B, _S, 56), jnp.float32),
        scratch_shapes=[pltpu.VMEM((_S, 4), jnp.float32)],
    )(masks, X, Wf)
    return out


# speedup vs baseline: 18.5313x; 1.9480x over previous
import jax
import jax.numpy as jnp
from jax.experimental import pallas as pl
from jax.experimental.pallas import tpu as pltpu

_S = 196
_H = 224
_W = 224
_HB = 32
_NH = _H // _HB
_KB = _HB * _W


def _sp_mean_kernel(mask_ref, xa_ref, wf_ref, out_ref, acc_ref):
    k = pl.program_id(1)

    @pl.when(k == 0)
    def _init():
        acc_ref[...] = jnp.zeros_like(acc_ref)

    m = mask_ref[0].reshape(_S, _KB).astype(jnp.float32)
    x3 = xa_ref[0].reshape(3, _KB)
    xa = jnp.concatenate([x3, jnp.ones((1, _KB), jnp.float32)], 0)
    acc_ref[...] += jax.lax.dot_general(
        m, xa, (((1,), (1,)), ((), ())), preferred_element_type=jnp.float32)

    @pl.when(k == _NH - 1)
    def _finish():
        acc = acc_ref[...]
        counts = acc[:, 3:4]
        proj = jax.lax.dot_general(
            acc, wf_ref[...], (((1,), (0,)), ((), ())),
            preferred_element_type=jnp.float32)
        out_ref[0] = proj / counts


def kernel(X, masks, W, b):
    B = X.shape[0]
    Wf = jnp.concatenate([W.T, b[None, :]], axis=0)

    out = pl.pallas_call(
        _sp_mean_kernel,
        grid=(B, _NH),
        in_specs=[
            pl.BlockSpec((1, _S, _HB, _W), lambda bi, ki: (bi, 0, ki, 0)),
            pl.BlockSpec((1, 3, _HB, _W), lambda bi, ki: (bi, 0, ki, 0)),
            pl.BlockSpec((4, 56), lambda bi, ki: (0, 0)),
        ],
        out_specs=pl.BlockSpec((1, _S, 56), lambda bi, ki: (bi, 0, 0)),
        out_shape=jax.ShapeDtypeStruct((B, _S, 56), jnp.float32),
        scratch_shapes=[pltpu.VMEM((_S, 4), jnp.float32)],
    )(masks.view(jnp.int8), X, Wf)
    return out


# speedup vs baseline: 18.8346x; 1.0164x over previous
import jax
import jax.numpy as jnp
from jax.experimental import pallas as pl
from jax.experimental.pallas import tpu as pltpu

_S = 196
_H = 224
_W = 224
_HB = 224
_NH = _H // _HB
_KB = _HB * _W


def _sp_mean_kernel(mask_ref, xa_ref, wf_ref, out_ref, acc_ref):
    k = pl.program_id(1)

    @pl.when(k == 0)
    def _init():
        acc_ref[...] = jnp.zeros_like(acc_ref)

    m = mask_ref[0].reshape(_S, _KB).astype(jnp.float32)
    x3 = xa_ref[0].reshape(3, _KB)
    xa = jnp.concatenate([x3, jnp.ones((1, _KB), jnp.float32)], 0)
    acc_ref[...] += jax.lax.dot_general(
        m, xa, (((1,), (1,)), ((), ())), preferred_element_type=jnp.float32)

    @pl.when(k == _NH - 1)
    def _finish():
        acc = acc_ref[...]
        counts = acc[:, 3:4]
        proj = jax.lax.dot_general(
            acc, wf_ref[...], (((1,), (0,)), ((), ())),
            preferred_element_type=jnp.float32)
        out_ref[0] = proj / counts


def kernel(X, masks, W, b):
    B = X.shape[0]
    Wf = jnp.concatenate([W.T, b[None, :]], axis=0)

    out = pl.pallas_call(
        _sp_mean_kernel,
        grid=(B, _NH),
        in_specs=[
            pl.BlockSpec((1, _S, _HB, _W), lambda bi, ki: (bi, 0, ki, 0)),
            pl.BlockSpec((1, 3, _HB, _W), lambda bi, ki: (bi, 0, ki, 0)),
            pl.BlockSpec((4, 56), lambda bi, ki: (0, 0)),
        ],
        out_specs=pl.BlockSpec((1, _S, 56), lambda bi, ki: (bi, 0, 0)),
        out_shape=jax.ShapeDtypeStruct((B, _S, 56), jnp.float32),
        scratch_shapes=[pltpu.VMEM((_S, 4), jnp.float32)],
    )(masks.view(jnp.int8), X, Wf)
    return out
